# Initial kernel scaffold; baseline (speedup 1.0000x reference)
import jax
import jax.numpy as jnp
from jax.experimental import pallas as pl


def kernel(x, edge_index, batch, W1_rel, W1_root, b1, W2, b2, W3, b3, Wlin, blin):
    raise NotImplementedError("write your pallas kernel here")


# speedup vs baseline: 8.8383x; 8.8383x over previous
import functools

import jax
import jax.numpy as jnp
from jax import lax
from jax.experimental import pallas as pl
from jax.experimental.pallas import tpu as pltpu
from jax.experimental.pallas import tpu_sc as plsc

NN = 10000
DD = 128
BB = 64
CC = 10

NC, NS = 2, 16
NW = NC * NS
CH = 128
NACC = 10240
RPT = NACC // NS
DEGR = NACC // DD
RB = 1000


def _spmm_body(with_deg, h, src, dst, *refs):
    if with_deg:
        out, degout, acc, deg_sh, zbuf, sidx, didx, rows, ones_v, z1, sem = refs
    else:
        out, acc, zbuf, sidx, didx, rows, sem = refs
    c = lax.axis_index("c")
    s = lax.axis_index("s")
    wid = s * NC + c
    nchunk = src.shape[0] // (NW * CH)
    epw = nchunk * CH

    def zrow(r, carry):
        for c8 in range(DD // 16):
            zbuf[r, pl.ds(c8 * 16, 16)] = jnp.zeros((16,), jnp.float32)
        return carry
    lax.fori_loop(0, CH, zrow, 0)

    for k in range(RPT // CH):
        acc_off = pl.multiple_of(s * RPT + k * CH, CH)
        pltpu.sync_copy(zbuf, acc.at[pl.ds(acc_off, CH)])

    if with_deg:
        def z1row(r, carry):
            z1[pl.ds(r * 16, 16)] = jnp.zeros((16,), jnp.float32)
            return carry
        lax.fori_loop(0, RPT // 16, z1row, 0)
        for j in range(CH // 16):
            ones_v[pl.ds(j * 16, 16)] = jnp.ones((16,), jnp.float32)
        deg_off = pl.multiple_of(s * RPT, RPT)
        pltpu.sync_copy(z1, deg_sh.at[pl.ds(deg_off, RPT)])

    plsc.subcore_barrier()

    def chunk(i, carry):
        base = pl.multiple_of(wid * epw + i * CH, CH)
        pltpu.sync_copy(src.at[pl.ds(base, CH)], sidx)
        pltpu.sync_copy(dst.at[pl.ds(base, CH)], didx)
        pltpu.async_copy(h.at[sidx], rows, sem).wait()
        pltpu.sync_copy(rows, acc.at[didx], add=True)
        if with_deg:
            pltpu.sync_copy(ones_v, deg_sh.at[didx], add=True)
        return carry
    lax.fori_loop(0, nchunk, chunk, 0)

    plsc.subcore_barrier()

    for k in range(RPT // CH):
        r0 = pl.multiple_of(s * RPT + k * CH, CH)
        pltpu.sync_copy(acc.at[pl.ds(r0, CH)], rows)
        pltpu.sync_copy(rows, out.at[c, pl.ds(r0, CH)])

    if with_deg:
        deg_off = pl.multiple_of(s * RPT, RPT)
        pltpu.sync_copy(deg_sh.at[pl.ds(deg_off, RPT)], z1)
        pltpu.sync_copy(z1, degout.at[c, pl.ds(deg_off, RPT)])


@functools.lru_cache(maxsize=4)
def _make_spmm(ep, with_deg):
    del ep
    out_type = [jax.ShapeDtypeStruct((NC, NACC, DD), jnp.float32)]
    scratch = [
        pltpu.VMEM_SHARED((NACC, DD), jnp.float32),
    ]
    if with_deg:
        out_type.append(jax.ShapeDtypeStruct((NC, NACC), jnp.float32))
        scratch.append(pltpu.VMEM_SHARED((NACC,), jnp.float32))
    scratch += [
        pltpu.VMEM((CH, DD), jnp.float32),
        pltpu.VMEM((CH,), jnp.int32),
        pltpu.VMEM((CH,), jnp.int32),
        pltpu.VMEM((CH, DD), jnp.float32),
    ]
    if with_deg:
        scratch += [
            pltpu.VMEM((CH,), jnp.float32),
            pltpu.VMEM((RPT,), jnp.float32),
        ]
    scratch.append(pltpu.SemaphoreType.DMA)
    mesh = plsc.VectorSubcoreMesh(core_axis_name="c", subcore_axis_name="s")
    return pl.kernel(
        functools.partial(_spmm_body, with_deg),
        out_type=tuple(out_type),
        mesh=mesh,
        scratch_types=scratch,
    )


def _matT(a, w):
    return lax.dot_general(a, w, (((1,), (1,)), ((), ())),
                           preferred_element_type=jnp.float32)


def _dis_body(dref, o):
    o[...] = lax.rsqrt(jnp.sum(dref[...], axis=0) + 1.0)


def _dense1_body(p0, p1, x, dis, w1r, w1o, b1, w2, o):
    aggr = p0[0] + p1[0]
    h1 = jnp.maximum(_matT(aggr, w1r[...]) + _matT(x[...], w1o[...]) + b1[...], 0.0)
    o[...] = dis[...] * _matT(h1, w2[...])


def _dense2_body(p0, p1, g, dis, b2, w3, o):
    h2 = jnp.maximum(dis[...] * (p0[0] + p1[0] + g[...]) + b2[...], 0.0)
    o[...] = dis[...] * _matT(h2, w3[...])


def _pool_body(p0, p1, g, dis, b3, bt, sums, cnts):
    i = pl.program_id(0)
    h3 = dis[...] * (p0[0] + p1[0] + g[...]) + b3[...]
    oh = (bt[...] == lax.broadcasted_iota(jnp.int32, (RB, BB), 1)
          ).astype(jnp.float32)
    s = lax.dot_general(oh, h3, (((0,), (0,)), ((), ())),
                        preferred_element_type=jnp.float32)
    cn = lax.dot_general(oh, jnp.ones((RB, DD), jnp.float32),
                         (((0,), (0,)), ((), ())),
                         preferred_element_type=jnp.float32)

    @pl.when(i == 0)
    def _():
        sums[...] = jnp.zeros_like(sums)
        cnts[...] = jnp.zeros_like(cnts)

    sums[...] += s
    cnts[...] += cn


def _head_body(sums, cnts, wl, bl, xn_ref, out_ref):
    pooled = sums[...] / jnp.maximum(cnts[...], 1.0)
    n2 = jnp.sum(pooled * pooled, axis=1, keepdims=True)
    xn = pooled * lax.rsqrt(jnp.maximum(n2, 1e-24))
    w = wl[...]
    wn2 = jnp.sum(w * w, axis=1, keepdims=True)
    wn = w * lax.rsqrt(jnp.maximum(wn2, 1e-24))
    xn_ref[...] = xn
    out_ref[...] = _matT(xn, wn) + bl[...]


def _row_specs(i_maps):
    return [pl.BlockSpec(shape, m) for shape, m in i_maps]


_GRID = NN // RB


def _dense1(part, x, dis, w1r, w1o, b1, w2):
    return pl.pallas_call(
        _dense1_body,
        grid=(_GRID,),
        in_specs=[
            pl.BlockSpec((1, RB, DD), lambda i: (0, i, 0)),
            pl.BlockSpec((1, RB, DD), lambda i: (1, i, 0)),
            pl.BlockSpec((RB, DD), lambda i: (i, 0)),
            pl.BlockSpec((RB, 1), lambda i: (i, 0)),
            pl.BlockSpec((DD, DD), lambda i: (0, 0)),
            pl.BlockSpec((DD, DD), lambda i: (0, 0)),
            pl.BlockSpec((1, DD), lambda i: (0, 0)),
            pl.BlockSpec((DD, DD), lambda i: (0, 0)),
        ],
        out_specs=pl.BlockSpec((RB, DD), lambda i: (i, 0)),
        out_shape=jax.ShapeDtypeStruct((NN, DD), jnp.float32),
    )(part, part, x, dis, w1r, w1o, b1, w2)


def _dense2(part, g, dis, b2, w3):
    return pl.pallas_call(
        _dense2_body,
        grid=(_GRID,),
        in_specs=[
            pl.BlockSpec((1, RB, DD), lambda i: (0, i, 0)),
            pl.BlockSpec((1, RB, DD), lambda i: (1, i, 0)),
            pl.BlockSpec((RB, DD), lambda i: (i, 0)),
            pl.BlockSpec((RB, 1), lambda i: (i, 0)),
            pl.BlockSpec((1, DD), lambda i: (0, 0)),
            pl.BlockSpec((DD, DD), lambda i: (0, 0)),
        ],
        out_specs=pl.BlockSpec((RB, DD), lambda i: (i, 0)),
        out_shape=jax.ShapeDtypeStruct((NN, DD), jnp.float32),
    )(part, part, g, dis, b2, w3)


def _pool(part, g, dis, b3, bt):
    return pl.pallas_call(
        _pool_body,
        grid=(_GRID,),
        in_specs=[
            pl.BlockSpec((1, RB, DD), lambda i: (0, i, 0)),
            pl.BlockSpec((1, RB, DD), lambda i: (1, i, 0)),
            pl.BlockSpec((RB, DD), lambda i: (i, 0)),
            pl.BlockSpec((RB, 1), lambda i: (i, 0)),
            pl.BlockSpec((1, DD), lambda i: (0, 0)),
            pl.BlockSpec((RB, 1), lambda i: (i, 0)),
        ],
        out_specs=[
            pl.BlockSpec((BB, DD), lambda i: (0, 0)),
            pl.BlockSpec((BB, DD), lambda i: (0, 0)),
        ],
        out_shape=[
            jax.ShapeDtypeStruct((BB, DD), jnp.float32),
            jax.ShapeDtypeStruct((BB, DD), jnp.float32),
        ],
    )(part, part, g, dis, b3, bt)


def _dis_call(degp):
    return pl.pallas_call(
        _dis_body,
        out_shape=jax.ShapeDtypeStruct((DEGR, DD), jnp.float32),
    )(degp)


def _head(sums, cnts, wlp, blp):
    return pl.pallas_call(
        _head_body,
        out_shape=[
            jax.ShapeDtypeStruct((BB, DD), jnp.float32),
            jax.ShapeDtypeStruct((BB, 16), jnp.float32),
        ],
    )(sums, cnts, wlp, blp)


def kernel(x, edge_index, batch, W1_rel, W1_root, b1, W2, b2, W3, b3, Wlin, blin):
    e = edge_index.shape[1]
    ep = -(-e // (NW * CH)) * (NW * CH)
    pad = ep - e
    src = jnp.concatenate([edge_index[0], jnp.zeros((pad,), jnp.int32)])
    dst = jnp.concatenate([edge_index[1], jnp.full((pad,), NN, jnp.int32)])

    spmm_deg = _make_spmm(ep, True)
    spmm = _make_spmm(ep, False)

    part1, degp = spmm_deg(x, src, dst)
    dis_img = _dis_call(degp.reshape(NC, DEGR, DD))
    dis = dis_img.reshape(-1)[:NN, None]

    b1r = b1.reshape(1, DD)
    b2r = b2.reshape(1, DD)
    b3r = b3.reshape(1, DD)

    g2 = _dense1(part1, x, dis, W1_rel, W1_root, b1r, W2)
    (part2,) = spmm(g2, src, dst)
    g3 = _dense2(part2, g2, dis, b2r, W3)
    (part3,) = spmm(g3, src, dst)

    bt = batch.reshape(NN, 1)
    sums, cnts = _pool(part3, g3, dis, b3r, bt)

    wlp = jnp.zeros((16, DD), jnp.float32).at[:CC].set(Wlin)
    blp = jnp.zeros((1, 16), jnp.float32).at[0, :CC].set(blin)
    xn, outp = _head(sums, cnts, wlp, blp)
    return (xn, outp[:, :CC])
